# Initial kernel scaffold; baseline (speedup 1.0000x reference)
import jax
import jax.numpy as jnp
from jax.experimental import pallas as pl


def kernel(x, edge_index, edge_attr, W_in, b_in, W_edge, b_edge, c0_eps, c0_W1, c0_b1, c0_g1, c0_be1, c0_W2, c0_b2, m0_W1, m0_b1, m0_g, m0_be, m0_W2, m0_b2, c1_eps, c1_W1, c1_b1, c1_g1, c1_be1, c1_W2, c1_b2, m1_W1, m1_b1, m1_g, m1_be, m1_W2, m1_b2, c2_eps, c2_W1, c2_b1, c2_g1, c2_be1, c2_W2, c2_b2, m2_W1, m2_b1, m2_g, m2_be, m2_W2, m2_b2, W_pool, b_pool):
    raise NotImplementedError("write your pallas kernel here")


# speedup vs baseline: 1.8781x; 1.8781x over previous
import functools

import jax
import jax.numpy as jnp
from jax import lax
from jax.experimental import pallas as pl
from jax.experimental.pallas import tpu as pltpu
from jax.experimental.pallas import tpu_sc as plsc

N = 10000
E = 320000
H = 256
HH = 128

NT = 16
K = 80
EPT = E // NT
NCH = EPT // K
STRIPE = 640
LAST = N - 15 * STRIPE

_PREC = jax.lax.Precision.HIGHEST


def _dot(a, b):
    return jax.lax.dot(a, b, precision=_PREC, preferred_element_type=jnp.float32)


def _bn_relu(t, g, b):
    m = jnp.mean(t, axis=0, keepdims=True)
    v = jnp.mean(t * t, axis=0, keepdims=True) - m * m
    t = (t - m) * jax.lax.rsqrt(v + 1e-5) * g + b
    return jnp.maximum(t, 0.0)


def _h_in_body(x_ref, w_ref, b_ref, o_ref):
    h = _dot(x_ref[...], w_ref[...]) + b_ref[...]
    o_ref[0] = h[:, :HH]
    o_ref[1] = h[:, HH:]


def _edge_body(ea_ref, w_ref, b_ref, o_ref):
    e = _dot(ea_ref[...], w_ref[...]) + b_ref[...]
    o_ref[0] = e[:, :HH]
    o_ref[1] = e[:, HH:]


def _t_body(h_ref, agg_ref, eps_ref, o_ref):
    o_ref[...] = (1.0 + eps_ref[0, 0]) * h_ref[...] + agg_ref[...]


def _mm_bn_body(t_ref, wa_ref, wb_ref, b_ref, g_ref, be_ref, o_ref):
    tarr = t_ref[...]
    z = _dot(tarr[:N], wa_ref[...]) + _dot(tarr[N:], wb_ref[...]) + b_ref[...]
    o_ref[...] = _bn_relu(z, g_ref[...], be_ref[...])


def _mm_split_body(z_ref, w_ref, b_ref, o_ref):
    zz = _dot(z_ref[...], w_ref[...]) + b_ref[...]
    o_ref[0] = zz[:, :HH]
    o_ref[1] = zz[:, HH:]


def _mm_res_body(z_ref, res_ref, w_ref, b_ref, o_ref, *, use_res):
    zz = _dot(z_ref[...], w_ref[...]) + b_ref[...]
    u0 = zz[:, :HH]
    u1 = zz[:, HH:]
    if use_res:
        u0 = u0 + res_ref[0]
        u1 = u1 + res_ref[1]
    o_ref[0] = jnp.where(u0 >= 0, u0, 0.1 * u0)
    o_ref[1] = jnp.where(u1 >= 0, u1, 0.1 * u1)


def _pool_body(h_ref, w_ref, b_ref, o_ref):
    harr = h_ref[...]
    h = jnp.concatenate([harr[:N], harr[N:]], axis=-1)
    hp = _dot(h, w_ref[...]) + b_ref[...]
    mx = jnp.max(hp, axis=0, keepdims=True)
    mn = jnp.mean(hp, axis=0, keepdims=True)
    o_ref[...] = jnp.concatenate([mx, mn], axis=-1)


def _sc_agg(h2, emb2, row, col2):
    mesh = plsc.VectorSubcoreMesh(core_axis_name="c", subcore_axis_name="s")

    @functools.partial(
        pl.kernel,
        out_type=jax.ShapeDtypeStruct((2 * N, HH), jnp.float32),
        mesh=mesh,
        scratch_types=[
            pltpu.VMEM((K,), jnp.int32),
            pltpu.VMEM((K,), jnp.int32),
            pltpu.VMEM((K, HH), jnp.float32),
            pltpu.VMEM((K, HH), jnp.float32),
            pltpu.VMEM((K, HH), jnp.float32),
            pltpu.VMEM_SHARED((N, HH), jnp.float32),
            pltpu.SemaphoreType.DMA,
        ],
    )
    def agg_kernel(h_hbm, emb_hbm, row_hbm, col_hbm, out_hbm,
                   colv, rowv, hrows, emb, msg, acc, sem):
        c = lax.axis_index("c")
        s = lax.axis_index("s")

        @pl.loop(0, K)
        def _(j):
            for i in range(HH // 16):
                msg[j, pl.ds(i * 16, 16)] = jnp.zeros((16,), jnp.float32)

        base_r = s * STRIPE

        @pl.when(s < 15)
        def _():
            for t in range(STRIPE // K):
                pltpu.sync_copy(msg, acc.at[pl.ds(base_r + t * K, K)])

        @pl.when(s == 15)
        def _():
            for t in range(LAST // K):
                pltpu.sync_copy(msg, acc.at[pl.ds(base_r + t * K, K)])

        plsc.subcore_barrier()

        @pl.loop(0, NCH)
        def _(ci):
            base = s * EPT + ci * K
            pltpu.sync_copy(col_hbm.at[pl.ds(c * E + base, K)], colv)
            pltpu.sync_copy(row_hbm.at[pl.ds(base, K)], rowv)
            pltpu.async_copy(h_hbm.at[colv], hrows, sem).wait()
            pltpu.sync_copy(emb_hbm.at[pl.ds(c * E + base, K)], emb)

            @pl.loop(0, K)
            def _(j):
                for i in range(HH // 16):
                    sl = pl.ds(i * 16, 16)
                    msg[j, sl] = hrows[j, sl] * emb[j, sl]

            pltpu.sync_copy(msg, acc.at[rowv], add=True)

        plsc.subcore_barrier()

        @pl.when(s < 15)
        def _():
            pltpu.sync_copy(acc.at[pl.ds(base_r, STRIPE)],
                            out_hbm.at[pl.ds(c * N + base_r, STRIPE)])

        @pl.when(s == 15)
        def _():
            pltpu.sync_copy(acc.at[pl.ds(base_r, LAST)],
                            out_hbm.at[pl.ds(c * N + base_r, LAST)])

    return agg_kernel(h2, emb2, row, col2)


def kernel(x, edge_index, edge_attr,
           W_in, b_in, W_edge, b_edge,
           c0_eps, c0_W1, c0_b1, c0_g1, c0_be1, c0_W2, c0_b2,
           m0_W1, m0_b1, m0_g, m0_be, m0_W2, m0_b2,
           c1_eps, c1_W1, c1_b1, c1_g1, c1_be1, c1_W2, c1_b2,
           m1_W1, m1_b1, m1_g, m1_be, m1_W2, m1_b2,
           c2_eps, c2_W1, c2_b1, c2_g1, c2_be1, c2_W2, c2_b2,
           m2_W1, m2_b1, m2_g, m2_be, m2_W2, m2_b2,
           W_pool, b_pool):
    f32 = jnp.float32
    row = edge_index[0]
    col = edge_index[1]
    col2 = jnp.concatenate([col, col + N])

    h2 = pl.pallas_call(
        _h_in_body,
        out_shape=jax.ShapeDtypeStruct((2, N, HH), f32),
    )(x, W_in, b_in.reshape(1, H)).reshape(2 * N, HH)

    EB = 8000
    emb2 = pl.pallas_call(
        _edge_body,
        grid=(E // EB,),
        in_specs=[
            pl.BlockSpec((EB, 9), lambda i: (i, 0)),
            pl.BlockSpec((9, H), lambda i: (0, 0)),
            pl.BlockSpec((1, H), lambda i: (0, 0)),
        ],
        out_specs=pl.BlockSpec((2, EB, HH), lambda i: (0, i, 0)),
        out_shape=jax.ShapeDtypeStruct((2, E, HH), f32),
    )(edge_attr, W_edge, b_edge.reshape(1, H)).reshape(2 * E, HH)

    layers = [
        (c0_eps, c0_W1, c0_b1, c0_g1, c0_be1, c0_W2, c0_b2,
         m0_W1, m0_b1, m0_g, m0_be, m0_W2, m0_b2),
        (c1_eps, c1_W1, c1_b1, c1_g1, c1_be1, c1_W2, c1_b2,
         m1_W1, m1_b1, m1_g, m1_be, m1_W2, m1_b2),
        (c2_eps, c2_W1, c2_b1, c2_g1, c2_be1, c2_W2, c2_b2,
         m2_W1, m2_b1, m2_g, m2_be, m2_W2, m2_b2),
    ]

    RB = 2000

    def mm_bn(t2, W, b, g, be):
        return pl.pallas_call(
            _mm_bn_body,
            grid=(H // HH,),
            in_specs=[
                pl.BlockSpec((2 * N, HH), lambda j: (0, 0)),
                pl.BlockSpec((HH, HH), lambda j: (0, j)),
                pl.BlockSpec((HH, HH), lambda j: (1, j)),
                pl.BlockSpec((1, HH), lambda j: (0, j)),
                pl.BlockSpec((1, HH), lambda j: (0, j)),
                pl.BlockSpec((1, HH), lambda j: (0, j)),
            ],
            out_specs=pl.BlockSpec((N, HH), lambda j: (0, j)),
            out_shape=jax.ShapeDtypeStruct((N, H), f32),
        )(t2, W, W, b.reshape(1, H), g.reshape(1, H), be.reshape(1, H))

    for i, (eps, W1, b1, g1, be1, W2, b2,
            mW1, mb1, mg, mbe, mW2, mb2) in enumerate(layers):
        agg2 = _sc_agg(h2, emb2, row, col2)
        t2 = pl.pallas_call(
            _t_body,
            grid=(2 * N // RB,),
            in_specs=[
                pl.BlockSpec((RB, HH), lambda r: (r, 0)),
                pl.BlockSpec((RB, HH), lambda r: (r, 0)),
                pl.BlockSpec((1, 1), lambda r: (0, 0)),
            ],
            out_specs=pl.BlockSpec((RB, HH), lambda r: (r, 0)),
            out_shape=jax.ShapeDtypeStruct((2 * N, HH), f32),
        )(h2, agg2, eps.reshape(1, 1))
        z = mm_bn(t2, W1, b1, g1, be1)
        h1 = pl.pallas_call(
            _mm_split_body,
            grid=(N // RB,),
            in_specs=[
                pl.BlockSpec((RB, H), lambda r: (r, 0)),
                pl.BlockSpec((H, H), lambda r: (0, 0)),
                pl.BlockSpec((1, H), lambda r: (0, 0)),
            ],
            out_specs=pl.BlockSpec((2, RB, HH), lambda r: (0, r, 0)),
            out_shape=jax.ShapeDtypeStruct((2, N, HH), f32),
        )(z, W2, b2.reshape(1, H)).reshape(2 * N, HH)
        u = mm_bn(h1, mW1, mb1, mg, mbe)
        h2 = pl.pallas_call(
            functools.partial(_mm_res_body, use_res=(i > 0)),
            grid=(N // RB,),
            in_specs=[
                pl.BlockSpec((RB, H), lambda r: (r, 0)),
                pl.BlockSpec((2, RB, HH), lambda r: (0, r, 0)),
                pl.BlockSpec((H, H), lambda r: (0, 0)),
                pl.BlockSpec((1, H), lambda r: (0, 0)),
            ],
            out_specs=pl.BlockSpec((2, RB, HH), lambda r: (0, r, 0)),
            out_shape=jax.ShapeDtypeStruct((2, N, HH), f32),
        )(u, h2.reshape(2, N, HH), mW2, mb2.reshape(1, H)).reshape(2 * N, HH)

    out = pl.pallas_call(
        _pool_body,
        out_shape=jax.ShapeDtypeStruct((1, 2 * H), f32),
    )(h2, W_pool, b_pool.reshape(1, H))
    return out


# speedup vs baseline: 3.2574x; 1.7345x over previous
import functools

import jax
import jax.numpy as jnp
from jax import lax
from jax.experimental import pallas as pl
from jax.experimental.pallas import tpu as pltpu
from jax.experimental.pallas import tpu_sc as plsc

N = 10000
E = 320000
H = 256
HH = 128

NT = 16
K = 40
EPT = E // NT
NCH = EPT // K
STRIPE = 640
LAST = N - 15 * STRIPE

_PREC = jax.lax.Precision.HIGHEST


def _dot(a, b):
    return jax.lax.dot(a, b, precision=_PREC, preferred_element_type=jnp.float32)


def _bn_relu(t, g, b):
    m = jnp.mean(t, axis=0, keepdims=True)
    v = jnp.mean(t * t, axis=0, keepdims=True) - m * m
    t = (t - m) * jax.lax.rsqrt(v + 1e-5) * g + b
    return jnp.maximum(t, 0.0)


def _h_in_body(x_ref, w_ref, b_ref, o_ref):
    h = _dot(x_ref[...], w_ref[...]) + b_ref[...]
    o_ref[0] = h[:, :HH]
    o_ref[1] = h[:, HH:]


def _edge_body(ea_ref, w_ref, b_ref, o_ref):
    e = _dot(ea_ref[...], w_ref[...]) + b_ref[...]
    o_ref[0] = e[:, :HH]
    o_ref[1] = e[:, HH:]


def _t_body(h_ref, agg_ref, eps_ref, o_ref):
    o_ref[...] = (1.0 + eps_ref[0, 0]) * h_ref[...] + agg_ref[...]


def _mm_bn_body(t_ref, wa_ref, wb_ref, b_ref, g_ref, be_ref, o_ref):
    tarr = t_ref[...]
    z = _dot(tarr[:N], wa_ref[...]) + _dot(tarr[N:], wb_ref[...]) + b_ref[...]
    o_ref[...] = _bn_relu(z, g_ref[...], be_ref[...])


def _mm_split_body(z_ref, w_ref, b_ref, o_ref):
    zz = _dot(z_ref[...], w_ref[...]) + b_ref[...]
    o_ref[0] = zz[:, :HH]
    o_ref[1] = zz[:, HH:]


def _mm_res_body(z_ref, res_ref, w_ref, b_ref, o_ref, *, use_res):
    zz = _dot(z_ref[...], w_ref[...]) + b_ref[...]
    u0 = zz[:, :HH]
    u1 = zz[:, HH:]
    if use_res:
        u0 = u0 + res_ref[0]
        u1 = u1 + res_ref[1]
    o_ref[0] = jnp.where(u0 >= 0, u0, 0.1 * u0)
    o_ref[1] = jnp.where(u1 >= 0, u1, 0.1 * u1)


def _pool_body(h_ref, w_ref, b_ref, o_ref):
    harr = h_ref[...]
    h = jnp.concatenate([harr[:N], harr[N:]], axis=-1)
    hp = _dot(h, w_ref[...]) + b_ref[...]
    mx = jnp.max(hp, axis=0, keepdims=True)
    mn = jnp.mean(hp, axis=0, keepdims=True)
    o_ref[...] = jnp.concatenate([mx, mn], axis=-1)


def _sc_agg(h2, emb2, row, col2):
    mesh = plsc.VectorSubcoreMesh(core_axis_name="c", subcore_axis_name="s")

    @functools.partial(
        pl.kernel,
        out_type=jax.ShapeDtypeStruct((2 * N, HH), jnp.float32),
        mesh=mesh,
        scratch_types=[
            pltpu.VMEM((2, K), jnp.int32),
            pltpu.VMEM((4, K), jnp.int32),
            pltpu.VMEM((2, K, HH), jnp.float32),
            pltpu.VMEM((2, K, HH), jnp.float32),
            pltpu.VMEM((2, K, HH), jnp.float32),
            pltpu.VMEM_SHARED((N, HH), jnp.float32),
            pltpu.SemaphoreType.DMA,
            pltpu.SemaphoreType.DMA,
            pltpu.SemaphoreType.DMA,
            pltpu.SemaphoreType.DMA,
            pltpu.SemaphoreType.DMA,
            pltpu.SemaphoreType.DMA,
            pltpu.SemaphoreType.DMA,
            pltpu.SemaphoreType.DMA,
        ],
    )
    def agg_kernel(h_hbm, emb_hbm, row_hbm, col_hbm, out_hbm,
                   colv, rowv, hr, em, ms, acc,
                   sg0, sg1, se0, se1, ss0, ss1, si0, si1):
        c = lax.axis_index("c")
        s = lax.axis_index("s")
        sg = (sg0, sg1)
        se = (se0, se1)
        ss = (ss0, ss1)
        si = (si0, si1)
        ioff = s * EPT
        eoff = c * E + s * EPT

        @pl.loop(0, K)
        def _(j):
            for i in range(HH // 16):
                hr[0, j, pl.ds(i * 16, 16)] = jnp.zeros((16,), jnp.float32)

        base_r = s * STRIPE

        @pl.when(s < 15)
        def _():
            for t in range(STRIPE // K):
                pltpu.sync_copy(hr.at[0], acc.at[pl.ds(base_r + t * K, K)])

        @pl.when(s == 15)
        def _():
            for t in range(LAST // K):
                pltpu.sync_copy(hr.at[0], acc.at[pl.ds(base_r + t * K, K)])

        def idx_fetch(ci, cslot, rslot, sem):
            pltpu.async_copy(col_hbm.at[pl.ds(c * E + ioff + ci * K, K)],
                             colv.at[cslot], sem)
            pltpu.async_copy(row_hbm.at[pl.ds(ioff + ci * K, K)],
                             rowv.at[rslot], sem)

        def idx_wait(cslot, rslot, sem):
            pltpu.make_async_copy(col_hbm.at[pl.ds(ioff, K)],
                                  colv.at[cslot], sem).wait()
            pltpu.make_async_copy(row_hbm.at[pl.ds(ioff, K)],
                                  rowv.at[rslot], sem).wait()

        def ge_fetch(ci, p):
            pltpu.async_copy(h_hbm.at[colv.at[p]], hr.at[p], sg[p])
            pltpu.async_copy(emb_hbm.at[pl.ds(eoff + ci * K, K)],
                             em.at[p], se[p])

        def ge_wait(ci, p):
            pltpu.make_async_copy(h_hbm.at[colv.at[p]], hr.at[p],
                                  sg[p]).wait()
            pltpu.make_async_copy(emb_hbm.at[pl.ds(eoff + ci * K, K)],
                                  em.at[p], se[p]).wait()

        idx_fetch(0, 0, 0, si[0])
        idx_wait(0, 0, si[0])
        idx_fetch(1, 1, 1, si[1])
        plsc.subcore_barrier()
        ge_fetch(0, 0)

        @pl.loop(0, NCH // 4)
        def _(cj):
            ci0 = cj * 4
            for q in range(4):
                ci = ci0 + q
                p = q % 2
                p1 = (p + 1) % 2

                @pl.when(ci >= 2)
                def _():
                    pltpu.make_async_copy(
                        ms.at[p], acc.at[rowv.at[(q + 2) % 4]],
                        ss[p]).wait()

                ge_wait(ci, p)

                @pl.when(ci + 2 < NCH)
                def _():
                    idx_fetch(ci + 2, p, (q + 2) % 4, si[p])

                @pl.when(ci + 1 < NCH)
                def _():
                    idx_wait(p1, (q + 1) % 4, si[p1])
                    ge_fetch(ci + 1, p1)

                @pl.loop(0, K)
                def _(j):
                    for i in range(HH // 16):
                        sl = pl.ds(i * 16, 16)
                        ms[p, j, sl] = hr[p, j, sl] * em[p, j, sl]

                pltpu.async_copy(ms.at[p], acc.at[rowv.at[q]], ss[p],
                                 add=True)

        pltpu.make_async_copy(ms.at[0], acc.at[rowv.at[2]], ss[0]).wait()
        pltpu.make_async_copy(ms.at[1], acc.at[rowv.at[3]], ss[1]).wait()

        plsc.subcore_barrier()

        @pl.when(s < 15)
        def _():
            pltpu.sync_copy(acc.at[pl.ds(base_r, STRIPE)],
                            out_hbm.at[pl.ds(c * N + base_r, STRIPE)])

        @pl.when(s == 15)
        def _():
            pltpu.sync_copy(acc.at[pl.ds(base_r, LAST)],
                            out_hbm.at[pl.ds(c * N + base_r, LAST)])

    return agg_kernel(h2, emb2, row, col2)


def kernel(x, edge_index, edge_attr,
           W_in, b_in, W_edge, b_edge,
           c0_eps, c0_W1, c0_b1, c0_g1, c0_be1, c0_W2, c0_b2,
           m0_W1, m0_b1, m0_g, m0_be, m0_W2, m0_b2,
           c1_eps, c1_W1, c1_b1, c1_g1, c1_be1, c1_W2, c1_b2,
           m1_W1, m1_b1, m1_g, m1_be, m1_W2, m1_b2,
           c2_eps, c2_W1, c2_b1, c2_g1, c2_be1, c2_W2, c2_b2,
           m2_W1, m2_b1, m2_g, m2_be, m2_W2, m2_b2,
           W_pool, b_pool):
    f32 = jnp.float32
    row = edge_index[0]
    col = edge_index[1]
    col2 = jnp.concatenate([col, col + N])

    h2 = pl.pallas_call(
        _h_in_body,
        out_shape=jax.ShapeDtypeStruct((2, N, HH), f32),
    )(x, W_in, b_in.reshape(1, H)).reshape(2 * N, HH)

    EB = 8000
    emb2 = pl.pallas_call(
        _edge_body,
        grid=(E // EB,),
        in_specs=[
            pl.BlockSpec((EB, 9), lambda i: (i, 0)),
            pl.BlockSpec((9, H), lambda i: (0, 0)),
            pl.BlockSpec((1, H), lambda i: (0, 0)),
        ],
        out_specs=pl.BlockSpec((2, EB, HH), lambda i: (0, i, 0)),
        out_shape=jax.ShapeDtypeStruct((2, E, HH), f32),
    )(edge_attr, W_edge, b_edge.reshape(1, H)).reshape(2 * E, HH)

    layers = [
        (c0_eps, c0_W1, c0_b1, c0_g1, c0_be1, c0_W2, c0_b2,
         m0_W1, m0_b1, m0_g, m0_be, m0_W2, m0_b2),
        (c1_eps, c1_W1, c1_b1, c1_g1, c1_be1, c1_W2, c1_b2,
         m1_W1, m1_b1, m1_g, m1_be, m1_W2, m1_b2),
        (c2_eps, c2_W1, c2_b1, c2_g1, c2_be1, c2_W2, c2_b2,
         m2_W1, m2_b1, m2_g, m2_be, m2_W2, m2_b2),
    ]

    RB = 2000

    def mm_bn(t2, W, b, g, be):
        return pl.pallas_call(
            _mm_bn_body,
            grid=(H // HH,),
            in_specs=[
                pl.BlockSpec((2 * N, HH), lambda j: (0, 0)),
                pl.BlockSpec((HH, HH), lambda j: (0, j)),
                pl.BlockSpec((HH, HH), lambda j: (1, j)),
                pl.BlockSpec((1, HH), lambda j: (0, j)),
                pl.BlockSpec((1, HH), lambda j: (0, j)),
                pl.BlockSpec((1, HH), lambda j: (0, j)),
            ],
            out_specs=pl.BlockSpec((N, HH), lambda j: (0, j)),
            out_shape=jax.ShapeDtypeStruct((N, H), f32),
        )(t2, W, W, b.reshape(1, H), g.reshape(1, H), be.reshape(1, H))

    for i, (eps, W1, b1, g1, be1, W2, b2,
            mW1, mb1, mg, mbe, mW2, mb2) in enumerate(layers):
        agg2 = _sc_agg(h2, emb2, row, col2)
        t2 = pl.pallas_call(
            _t_body,
            grid=(2 * N // RB,),
            in_specs=[
                pl.BlockSpec((RB, HH), lambda r: (r, 0)),
                pl.BlockSpec((RB, HH), lambda r: (r, 0)),
                pl.BlockSpec((1, 1), lambda r: (0, 0)),
            ],
            out_specs=pl.BlockSpec((RB, HH), lambda r: (r, 0)),
            out_shape=jax.ShapeDtypeStruct((2 * N, HH), f32),
        )(h2, agg2, eps.reshape(1, 1))
        z = mm_bn(t2, W1, b1, g1, be1)
        h1 = pl.pallas_call(
            _mm_split_body,
            grid=(N // RB,),
            in_specs=[
                pl.BlockSpec((RB, H), lambda r: (r, 0)),
                pl.BlockSpec((H, H), lambda r: (0, 0)),
                pl.BlockSpec((1, H), lambda r: (0, 0)),
            ],
            out_specs=pl.BlockSpec((2, RB, HH), lambda r: (0, r, 0)),
            out_shape=jax.ShapeDtypeStruct((2, N, HH), f32),
        )(z, W2, b2.reshape(1, H)).reshape(2 * N, HH)
        u = mm_bn(h1, mW1, mb1, mg, mbe)
        h2 = pl.pallas_call(
            functools.partial(_mm_res_body, use_res=(i > 0)),
            grid=(N // RB,),
            in_specs=[
                pl.BlockSpec((RB, H), lambda r: (r, 0)),
                pl.BlockSpec((2, RB, HH), lambda r: (0, r, 0)),
                pl.BlockSpec((H, H), lambda r: (0, 0)),
                pl.BlockSpec((1, H), lambda r: (0, 0)),
            ],
            out_specs=pl.BlockSpec((2, RB, HH), lambda r: (0, r, 0)),
            out_shape=jax.ShapeDtypeStruct((2, N, HH), f32),
        )(u, h2.reshape(2, N, HH), mW2, mb2.reshape(1, H)).reshape(2 * N, HH)

    out = pl.pallas_call(
        _pool_body,
        out_shape=jax.ShapeDtypeStruct((1, 2 * H), f32),
    )(h2, W_pool, b_pool.reshape(1, H))
    return out


# speedup vs baseline: 3.6582x; 1.1230x over previous
import functools

import jax
import jax.numpy as jnp
from jax import lax
from jax.experimental import pallas as pl
from jax.experimental.pallas import tpu as pltpu
from jax.experimental.pallas import tpu_sc as plsc

N = 10000
E = 320000
H = 256
HH = 128

NT = 16
K = 40
EPT = E // NT
NCH = EPT // K
STRIPE = 640
LAST = N - 15 * STRIPE

_PREC = jax.lax.Precision.DEFAULT


def _dot(a, b):
    return jax.lax.dot(a, b, precision=_PREC, preferred_element_type=jnp.float32)


def _bn_relu(t, g, b):
    m = jnp.mean(t, axis=0, keepdims=True)
    v = jnp.mean(t * t, axis=0, keepdims=True) - m * m
    t = (t - m) * jax.lax.rsqrt(v + 1e-5) * g + b
    return jnp.maximum(t, 0.0)


def _h_in_body(x_ref, w_ref, b_ref, o_ref):
    h = _dot(x_ref[...], w_ref[...]) + b_ref[...]
    o_ref[0] = h[:, :HH]
    o_ref[1] = h[:, HH:]


def _edge_body(ea_ref, w_ref, b_ref, o_ref):
    e = _dot(ea_ref[...], w_ref[...]) + b_ref[...]
    o_ref[0] = e[:, :HH]
    o_ref[1] = e[:, HH:]


def _t_body(h_ref, agg_ref, eps_ref, o_ref):
    o_ref[...] = (1.0 + eps_ref[0, 0]) * h_ref[...] + agg_ref[...]


def _mm_bn_body(t_ref, wa_ref, wb_ref, b_ref, g_ref, be_ref, o_ref):
    tarr = t_ref[...]
    z = _dot(tarr[:N], wa_ref[...]) + _dot(tarr[N:], wb_ref[...]) + b_ref[...]
    o_ref[...] = _bn_relu(z, g_ref[...], be_ref[...])


def _mm_split_body(z_ref, w_ref, b_ref, o_ref):
    zz = _dot(z_ref[...], w_ref[...]) + b_ref[...]
    o_ref[0] = zz[:, :HH]
    o_ref[1] = zz[:, HH:]


def _mm_res_body(z_ref, res_ref, w_ref, b_ref, o_ref, *, use_res):
    zz = _dot(z_ref[...], w_ref[...]) + b_ref[...]
    u0 = zz[:, :HH]
    u1 = zz[:, HH:]
    if use_res:
        u0 = u0 + res_ref[0]
        u1 = u1 + res_ref[1]
    o_ref[0] = jnp.where(u0 >= 0, u0, 0.1 * u0)
    o_ref[1] = jnp.where(u1 >= 0, u1, 0.1 * u1)


def _pool_body(h_ref, w_ref, b_ref, o_ref):
    harr = h_ref[...]
    h = jnp.concatenate([harr[:N], harr[N:]], axis=-1)
    hp = _dot(h, w_ref[...]) + b_ref[...]
    mx = jnp.max(hp, axis=0, keepdims=True)
    mn = jnp.mean(hp, axis=0, keepdims=True)
    o_ref[...] = jnp.concatenate([mx, mn], axis=-1)


def _sc_agg(h2, emb2, row, col2):
    mesh = plsc.VectorSubcoreMesh(core_axis_name="c", subcore_axis_name="s")

    @functools.partial(
        pl.kernel,
        out_type=jax.ShapeDtypeStruct((2 * N, HH), jnp.float32),
        mesh=mesh,
        scratch_types=[
            pltpu.VMEM((2, K), jnp.int32),
            pltpu.VMEM((4, K), jnp.int32),
            pltpu.VMEM((2, K, HH), jnp.float32),
            pltpu.VMEM((2, K, HH), jnp.float32),
            pltpu.VMEM((2, K, HH), jnp.float32),
            pltpu.VMEM_SHARED((N, HH), jnp.float32),
            pltpu.SemaphoreType.DMA,
            pltpu.SemaphoreType.DMA,
            pltpu.SemaphoreType.DMA,
            pltpu.SemaphoreType.DMA,
            pltpu.SemaphoreType.DMA,
            pltpu.SemaphoreType.DMA,
            pltpu.SemaphoreType.DMA,
            pltpu.SemaphoreType.DMA,
        ],
    )
    def agg_kernel(h_hbm, emb_hbm, row_hbm, col_hbm, out_hbm,
                   colv, rowv, hr, em, ms, acc,
                   sg0, sg1, se0, se1, ss0, ss1, si0, si1):
        c = lax.axis_index("c")
        s = lax.axis_index("s")
        sg = (sg0, sg1)
        se = (se0, se1)
        ss = (ss0, ss1)
        si = (si0, si1)
        ioff = s * EPT
        eoff = c * E + s * EPT

        @pl.loop(0, K)
        def _(j):
            for i in range(HH // 16):
                hr[0, j, pl.ds(i * 16, 16)] = jnp.zeros((16,), jnp.float32)

        base_r = s * STRIPE

        @pl.when(s < 15)
        def _():
            for t in range(STRIPE // K):
                pltpu.sync_copy(hr.at[0], acc.at[pl.ds(base_r + t * K, K)])

        @pl.when(s == 15)
        def _():
            for t in range(LAST // K):
                pltpu.sync_copy(hr.at[0], acc.at[pl.ds(base_r + t * K, K)])

        def idx_fetch(ci, cslot, rslot, sem):
            pltpu.async_copy(col_hbm.at[pl.ds(c * E + ioff + ci * K, K)],
                             colv.at[cslot], sem)
            pltpu.async_copy(row_hbm.at[pl.ds(ioff + ci * K, K)],
                             rowv.at[rslot], sem)

        def idx_wait(cslot, rslot, sem):
            pltpu.make_async_copy(col_hbm.at[pl.ds(ioff, K)],
                                  colv.at[cslot], sem).wait()
            pltpu.make_async_copy(row_hbm.at[pl.ds(ioff, K)],
                                  rowv.at[rslot], sem).wait()

        def ge_fetch(ci, p):
            pltpu.async_copy(h_hbm.at[colv.at[p]], hr.at[p], sg[p])
            pltpu.async_copy(emb_hbm.at[pl.ds(eoff + ci * K, K)],
                             em.at[p], se[p])

        def ge_wait(ci, p):
            pltpu.make_async_copy(h_hbm.at[colv.at[p]], hr.at[p],
                                  sg[p]).wait()
            pltpu.make_async_copy(emb_hbm.at[pl.ds(eoff + ci * K, K)],
                                  em.at[p], se[p]).wait()

        idx_fetch(0, 0, 0, si[0])
        idx_wait(0, 0, si[0])
        idx_fetch(1, 1, 1, si[1])
        plsc.subcore_barrier()
        ge_fetch(0, 0)

        @pl.loop(0, NCH // 4)
        def _(cj):
            ci0 = cj * 4
            for q in range(4):
                ci = ci0 + q
                p = q % 2
                p1 = (p + 1) % 2

                @pl.when(ci >= 2)
                def _():
                    pltpu.make_async_copy(
                        ms.at[p], acc.at[rowv.at[(q + 2) % 4]],
                        ss[p]).wait()

                ge_wait(ci, p)

                @pl.when(ci + 2 < NCH)
                def _():
                    idx_fetch(ci + 2, p, (q + 2) % 4, si[p])

                @pl.when(ci + 1 < NCH)
                def _():
                    idx_wait(p1, (q + 1) % 4, si[p1])
                    ge_fetch(ci + 1, p1)

                @pl.loop(0, K)
                def _(j):
                    for i in range(HH // 16):
                        sl = pl.ds(i * 16, 16)
                        ms[p, j, sl] = hr[p, j, sl] * em[p, j, sl]

                pltpu.async_copy(ms.at[p], acc.at[rowv.at[q]], ss[p],
                                 add=True)

        pltpu.make_async_copy(ms.at[0], acc.at[rowv.at[2]], ss[0]).wait()
        pltpu.make_async_copy(ms.at[1], acc.at[rowv.at[3]], ss[1]).wait()

        plsc.subcore_barrier()

        @pl.when(s < 15)
        def _():
            pltpu.sync_copy(acc.at[pl.ds(base_r, STRIPE)],
                            out_hbm.at[pl.ds(c * N + base_r, STRIPE)])

        @pl.when(s == 15)
        def _():
            pltpu.sync_copy(acc.at[pl.ds(base_r, LAST)],
                            out_hbm.at[pl.ds(c * N + base_r, LAST)])

    return agg_kernel(h2, emb2, row, col2)


def kernel(x, edge_index, edge_attr,
           W_in, b_in, W_edge, b_edge,
           c0_eps, c0_W1, c0_b1, c0_g1, c0_be1, c0_W2, c0_b2,
           m0_W1, m0_b1, m0_g, m0_be, m0_W2, m0_b2,
           c1_eps, c1_W1, c1_b1, c1_g1, c1_be1, c1_W2, c1_b2,
           m1_W1, m1_b1, m1_g, m1_be, m1_W2, m1_b2,
           c2_eps, c2_W1, c2_b1, c2_g1, c2_be1, c2_W2, c2_b2,
           m2_W1, m2_b1, m2_g, m2_be, m2_W2, m2_b2,
           W_pool, b_pool):
    f32 = jnp.float32
    row = edge_index[0]
    col = edge_index[1]
    col2 = jnp.concatenate([col, col + N])

    h2 = pl.pallas_call(
        _h_in_body,
        out_shape=jax.ShapeDtypeStruct((2, N, HH), f32),
    )(x, W_in, b_in.reshape(1, H)).reshape(2 * N, HH)

    EB = 8000
    emb2 = pl.pallas_call(
        _edge_body,
        grid=(E // EB,),
        in_specs=[
            pl.BlockSpec((EB, 9), lambda i: (i, 0)),
            pl.BlockSpec((9, H), lambda i: (0, 0)),
            pl.BlockSpec((1, H), lambda i: (0, 0)),
        ],
        out_specs=pl.BlockSpec((2, EB, HH), lambda i: (0, i, 0)),
        out_shape=jax.ShapeDtypeStruct((2, E, HH), f32),
    )(edge_attr, W_edge, b_edge.reshape(1, H)).reshape(2 * E, HH)

    layers = [
        (c0_eps, c0_W1, c0_b1, c0_g1, c0_be1, c0_W2, c0_b2,
         m0_W1, m0_b1, m0_g, m0_be, m0_W2, m0_b2),
        (c1_eps, c1_W1, c1_b1, c1_g1, c1_be1, c1_W2, c1_b2,
         m1_W1, m1_b1, m1_g, m1_be, m1_W2, m1_b2),
        (c2_eps, c2_W1, c2_b1, c2_g1, c2_be1, c2_W2, c2_b2,
         m2_W1, m2_b1, m2_g, m2_be, m2_W2, m2_b2),
    ]

    RB = 2000

    def mm_bn(t2, W, b, g, be):
        return pl.pallas_call(
            _mm_bn_body,
            grid=(H // HH,),
            in_specs=[
                pl.BlockSpec((2 * N, HH), lambda j: (0, 0)),
                pl.BlockSpec((HH, HH), lambda j: (0, j)),
                pl.BlockSpec((HH, HH), lambda j: (1, j)),
                pl.BlockSpec((1, HH), lambda j: (0, j)),
                pl.BlockSpec((1, HH), lambda j: (0, j)),
                pl.BlockSpec((1, HH), lambda j: (0, j)),
            ],
            out_specs=pl.BlockSpec((N, HH), lambda j: (0, j)),
            out_shape=jax.ShapeDtypeStruct((N, H), f32),
        )(t2, W, W, b.reshape(1, H), g.reshape(1, H), be.reshape(1, H))

    for i, (eps, W1, b1, g1, be1, W2, b2,
            mW1, mb1, mg, mbe, mW2, mb2) in enumerate(layers):
        agg2 = _sc_agg(h2, emb2, row, col2)
        t2 = pl.pallas_call(
            _t_body,
            grid=(2 * N // RB,),
            in_specs=[
                pl.BlockSpec((RB, HH), lambda r: (r, 0)),
                pl.BlockSpec((RB, HH), lambda r: (r, 0)),
                pl.BlockSpec((1, 1), lambda r: (0, 0)),
            ],
            out_specs=pl.BlockSpec((RB, HH), lambda r: (r, 0)),
            out_shape=jax.ShapeDtypeStruct((2 * N, HH), f32),
        )(h2, agg2, eps.reshape(1, 1))
        z = mm_bn(t2, W1, b1, g1, be1)
        h1 = pl.pallas_call(
            _mm_split_body,
            grid=(N // RB,),
            in_specs=[
                pl.BlockSpec((RB, H), lambda r: (r, 0)),
                pl.BlockSpec((H, H), lambda r: (0, 0)),
                pl.BlockSpec((1, H), lambda r: (0, 0)),
            ],
            out_specs=pl.BlockSpec((2, RB, HH), lambda r: (0, r, 0)),
            out_shape=jax.ShapeDtypeStruct((2, N, HH), f32),
        )(z, W2, b2.reshape(1, H)).reshape(2 * N, HH)
        u = mm_bn(h1, mW1, mb1, mg, mbe)
        h2 = pl.pallas_call(
            functools.partial(_mm_res_body, use_res=(i > 0)),
            grid=(N // RB,),
            in_specs=[
                pl.BlockSpec((RB, H), lambda r: (r, 0)),
                pl.BlockSpec((2, RB, HH), lambda r: (0, r, 0)),
                pl.BlockSpec((H, H), lambda r: (0, 0)),
                pl.BlockSpec((1, H), lambda r: (0, 0)),
            ],
            out_specs=pl.BlockSpec((2, RB, HH), lambda r: (0, r, 0)),
            out_shape=jax.ShapeDtypeStruct((2, N, HH), f32),
        )(u, h2.reshape(2, N, HH), mW2, mb2.reshape(1, H)).reshape(2 * N, HH)

    out = pl.pallas_call(
        _pool_body,
        out_shape=jax.ShapeDtypeStruct((1, 2 * H), f32),
    )(h2, W_pool, b_pool.reshape(1, H))
    return out


# speedup vs baseline: 3.7122x; 1.0148x over previous
import functools

import jax
import jax.numpy as jnp
from jax import lax
from jax.experimental import pallas as pl
from jax.experimental.pallas import tpu as pltpu
from jax.experimental.pallas import tpu_sc as plsc

N = 10000
E = 320000
H = 256
HH = 128

NT = 16
K = 40
EPT = E // NT
NCH = EPT // K
STRIPE = 640
LAST = N - 15 * STRIPE

_PREC = jax.lax.Precision.DEFAULT


def _dot(a, b):
    return jax.lax.dot(a, b, precision=_PREC, preferred_element_type=jnp.float32)


def _bn_relu(t, g, b):
    m = jnp.mean(t, axis=0, keepdims=True)
    v = jnp.mean(t * t, axis=0, keepdims=True) - m * m
    t = (t - m) * jax.lax.rsqrt(v + 1e-5) * g + b
    return jnp.maximum(t, 0.0)


def _h_in_body(x_ref, w_ref, b_ref, o_ref):
    h = _dot(x_ref[...], w_ref[...]) + b_ref[...]
    o_ref[0] = h[:, :HH]
    o_ref[1] = h[:, HH:]


def _edge_body(ea_ref, w_ref, b_ref, o_ref):
    e = _dot(ea_ref[...], w_ref[...]) + b_ref[...]
    o_ref[0] = e[:, :HH]
    o_ref[1] = e[:, HH:]


def _conv1_body(h_ref, agg_ref, eps_ref, wa_ref, wb_ref, b_ref, g_ref,
                be_ref, o_ref):
    sc = 1.0 + eps_ref[0, 0]
    harr = h_ref[...]
    aarr = agg_ref[...]
    t0 = sc * harr[:N] + aarr[:N]
    t1 = sc * harr[N:] + aarr[N:]
    z = _dot(t0, wa_ref[...]) + _dot(t1, wb_ref[...]) + b_ref[...]
    o_ref[...] = _bn_relu(z, g_ref[...], be_ref[...])


def _mm_bn_body(t_ref, wa_ref, wb_ref, b_ref, g_ref, be_ref, o_ref):
    tarr = t_ref[...]
    z = _dot(tarr[:N], wa_ref[...]) + _dot(tarr[N:], wb_ref[...]) + b_ref[...]
    o_ref[...] = _bn_relu(z, g_ref[...], be_ref[...])


def _mm_split_body(z_ref, w_ref, b_ref, o_ref):
    zz = _dot(z_ref[...], w_ref[...]) + b_ref[...]
    o_ref[0] = zz[:, :HH]
    o_ref[1] = zz[:, HH:]


def _mm_res_body(z_ref, res_ref, w_ref, b_ref, o_ref, *, use_res):
    zz = _dot(z_ref[...], w_ref[...]) + b_ref[...]
    u0 = zz[:, :HH]
    u1 = zz[:, HH:]
    if use_res:
        u0 = u0 + res_ref[0]
        u1 = u1 + res_ref[1]
    o_ref[0] = jnp.where(u0 >= 0, u0, 0.1 * u0)
    o_ref[1] = jnp.where(u1 >= 0, u1, 0.1 * u1)


def _pool_body(h_ref, w_ref, b_ref, o_ref):
    harr = h_ref[...]
    h = jnp.concatenate([harr[:N], harr[N:]], axis=-1)
    hp = _dot(h, w_ref[...]) + b_ref[...]
    mx = jnp.max(hp, axis=0, keepdims=True)
    mn = jnp.mean(hp, axis=0, keepdims=True)
    o_ref[...] = jnp.concatenate([mx, mn], axis=-1)


def _sc_agg(h2, emb2, row, col2):
    mesh = plsc.VectorSubcoreMesh(core_axis_name="c", subcore_axis_name="s")

    @functools.partial(
        pl.kernel,
        out_type=jax.ShapeDtypeStruct((2 * N, HH), jnp.float32),
        mesh=mesh,
        scratch_types=[
            pltpu.VMEM((2, K), jnp.int32),
            pltpu.VMEM((4, K), jnp.int32),
            pltpu.VMEM((2, K, HH), jnp.float32),
            pltpu.VMEM((2, K, HH), jnp.float32),
            pltpu.VMEM((2, K, HH), jnp.float32),
            pltpu.VMEM_SHARED((N, HH), jnp.float32),
            pltpu.SemaphoreType.DMA,
            pltpu.SemaphoreType.DMA,
            pltpu.SemaphoreType.DMA,
            pltpu.SemaphoreType.DMA,
            pltpu.SemaphoreType.DMA,
            pltpu.SemaphoreType.DMA,
            pltpu.SemaphoreType.DMA,
            pltpu.SemaphoreType.DMA,
        ],
    )
    def agg_kernel(h_hbm, emb_hbm, row_hbm, col_hbm, out_hbm,
                   colv, rowv, hr, em, ms, acc,
                   sg0, sg1, se0, se1, ss0, ss1, si0, si1):
        c = lax.axis_index("c")
        s = lax.axis_index("s")
        sg = (sg0, sg1)
        se = (se0, se1)
        ss = (ss0, ss1)
        si = (si0, si1)
        ioff = s * EPT
        eoff = c * E + s * EPT

        @pl.loop(0, K)
        def _(j):
            for i in range(HH // 16):
                hr[0, j, pl.ds(i * 16, 16)] = jnp.zeros((16,), jnp.float32)

        base_r = s * STRIPE

        @pl.when(s < 15)
        def _():
            for t in range(STRIPE // K):
                pltpu.async_copy(hr.at[0], acc.at[pl.ds(base_r + t * K, K)],
                                 sg0)

        @pl.when(s == 15)
        def _():
            for t in range(LAST // K):
                pltpu.async_copy(hr.at[0], acc.at[pl.ds(base_r + t * K, K)],
                                 sg0)

        def idx_fetch(ci, cslot, rslot, sem):
            pltpu.async_copy(col_hbm.at[pl.ds(c * E + ioff + ci * K, K)],
                             colv.at[cslot], sem)
            pltpu.async_copy(row_hbm.at[pl.ds(ioff + ci * K, K)],
                             rowv.at[rslot], sem)

        def idx_wait(cslot, rslot, sem):
            pltpu.make_async_copy(col_hbm.at[pl.ds(ioff, K)],
                                  colv.at[cslot], sem).wait()
            pltpu.make_async_copy(row_hbm.at[pl.ds(ioff, K)],
                                  rowv.at[rslot], sem).wait()

        def ge_fetch(ci, p):
            pltpu.async_copy(h_hbm.at[colv.at[p]], hr.at[p], sg[p])
            pltpu.async_copy(emb_hbm.at[pl.ds(eoff + ci * K, K)],
                             em.at[p], se[p])

        def ge_wait(ci, p):
            pltpu.make_async_copy(h_hbm.at[colv.at[p]], hr.at[p],
                                  sg[p]).wait()
            pltpu.make_async_copy(emb_hbm.at[pl.ds(eoff + ci * K, K)],
                                  em.at[p], se[p]).wait()

        idx_fetch(0, 0, 0, si[0])
        idx_fetch(1, 1, 1, si[1])

        @pl.when(s < 15)
        def _():
            for t in range(STRIPE // K):
                pltpu.make_async_copy(
                    hr.at[0], acc.at[pl.ds(base_r + t * K, K)], sg0).wait()

        @pl.when(s == 15)
        def _():
            for t in range(LAST // K):
                pltpu.make_async_copy(
                    hr.at[0], acc.at[pl.ds(base_r + t * K, K)], sg0).wait()

        idx_wait(0, 0, si[0])
        plsc.subcore_barrier()
        ge_fetch(0, 0)

        @pl.loop(0, NCH // 4)
        def _(cj):
            ci0 = cj * 4
            for q in range(4):
                ci = ci0 + q
                p = q % 2
                p1 = (p + 1) % 2

                @pl.when(ci >= 2)
                def _():
                    pltpu.make_async_copy(
                        ms.at[p], acc.at[rowv.at[(q + 2) % 4]],
                        ss[p]).wait()

                ge_wait(ci, p)

                @pl.when(ci + 2 < NCH)
                def _():
                    idx_fetch(ci + 2, p, (q + 2) % 4, si[p])

                @pl.when(ci + 1 < NCH)
                def _():
                    idx_wait(p1, (q + 1) % 4, si[p1])
                    ge_fetch(ci + 1, p1)

                @plsc.parallel_loop(0, K, unroll=4)
                def _(j):
                    for i in range(HH // 16):
                        sl = pl.ds(i * 16, 16)
                        ms[p, j, sl] = hr[p, j, sl] * em[p, j, sl]

                pltpu.async_copy(ms.at[p], acc.at[rowv.at[q]], ss[p],
                                 add=True)

        pltpu.make_async_copy(ms.at[0], acc.at[rowv.at[2]], ss[0]).wait()
        pltpu.make_async_copy(ms.at[1], acc.at[rowv.at[3]], ss[1]).wait()

        plsc.subcore_barrier()

        @pl.when(s < 15)
        def _():
            pltpu.sync_copy(acc.at[pl.ds(base_r, STRIPE)],
                            out_hbm.at[pl.ds(c * N + base_r, STRIPE)])

        @pl.when(s == 15)
        def _():
            pltpu.sync_copy(acc.at[pl.ds(base_r, LAST)],
                            out_hbm.at[pl.ds(c * N + base_r, LAST)])

    return agg_kernel(h2, emb2, row, col2)


def kernel(x, edge_index, edge_attr,
           W_in, b_in, W_edge, b_edge,
           c0_eps, c0_W1, c0_b1, c0_g1, c0_be1, c0_W2, c0_b2,
           m0_W1, m0_b1, m0_g, m0_be, m0_W2, m0_b2,
           c1_eps, c1_W1, c1_b1, c1_g1, c1_be1, c1_W2, c1_b2,
           m1_W1, m1_b1, m1_g, m1_be, m1_W2, m1_b2,
           c2_eps, c2_W1, c2_b1, c2_g1, c2_be1, c2_W2, c2_b2,
           m2_W1, m2_b1, m2_g, m2_be, m2_W2, m2_b2,
           W_pool, b_pool):
    f32 = jnp.float32
    row = edge_index[0]
    col = edge_index[1]
    col2 = jnp.concatenate([col, col + N])

    h2 = pl.pallas_call(
        _h_in_body,
        out_shape=jax.ShapeDtypeStruct((2, N, HH), f32),
    )(x, W_in, b_in.reshape(1, H)).reshape(2 * N, HH)

    EB = 8000
    emb2 = pl.pallas_call(
        _edge_body,
        grid=(E // EB,),
        in_specs=[
            pl.BlockSpec((EB, 9), lambda i: (i, 0)),
            pl.BlockSpec((9, H), lambda i: (0, 0)),
            pl.BlockSpec((1, H), lambda i: (0, 0)),
        ],
        out_specs=pl.BlockSpec((2, EB, HH), lambda i: (0, i, 0)),
        out_shape=jax.ShapeDtypeStruct((2, E, HH), f32),
    )(edge_attr, W_edge, b_edge.reshape(1, H)).reshape(2 * E, HH)

    layers = [
        (c0_eps, c0_W1, c0_b1, c0_g1, c0_be1, c0_W2, c0_b2,
         m0_W1, m0_b1, m0_g, m0_be, m0_W2, m0_b2),
        (c1_eps, c1_W1, c1_b1, c1_g1, c1_be1, c1_W2, c1_b2,
         m1_W1, m1_b1, m1_g, m1_be, m1_W2, m1_b2),
        (c2_eps, c2_W1, c2_b1, c2_g1, c2_be1, c2_W2, c2_b2,
         m2_W1, m2_b1, m2_g, m2_be, m2_W2, m2_b2),
    ]

    RB = 2000

    def mm_bn(t2, W, b, g, be):
        return pl.pallas_call(
            _mm_bn_body,
            grid=(H // HH,),
            in_specs=[
                pl.BlockSpec((2 * N, HH), lambda j: (0, 0)),
                pl.BlockSpec((HH, HH), lambda j: (0, j)),
                pl.BlockSpec((HH, HH), lambda j: (1, j)),
                pl.BlockSpec((1, HH), lambda j: (0, j)),
                pl.BlockSpec((1, HH), lambda j: (0, j)),
                pl.BlockSpec((1, HH), lambda j: (0, j)),
            ],
            out_specs=pl.BlockSpec((N, HH), lambda j: (0, j)),
            out_shape=jax.ShapeDtypeStruct((N, H), f32),
        )(t2, W, W, b.reshape(1, H), g.reshape(1, H), be.reshape(1, H))

    for i, (eps, W1, b1, g1, be1, W2, b2,
            mW1, mb1, mg, mbe, mW2, mb2) in enumerate(layers):
        agg2 = _sc_agg(h2, emb2, row, col2)
        z = pl.pallas_call(
            _conv1_body,
            grid=(H // HH,),
            in_specs=[
                pl.BlockSpec((2 * N, HH), lambda j: (0, 0)),
                pl.BlockSpec((2 * N, HH), lambda j: (0, 0)),
                pl.BlockSpec((1, 1), lambda j: (0, 0)),
                pl.BlockSpec((HH, HH), lambda j: (0, j)),
                pl.BlockSpec((HH, HH), lambda j: (1, j)),
                pl.BlockSpec((1, HH), lambda j: (0, j)),
                pl.BlockSpec((1, HH), lambda j: (0, j)),
                pl.BlockSpec((1, HH), lambda j: (0, j)),
            ],
            out_specs=pl.BlockSpec((N, HH), lambda j: (0, j)),
            out_shape=jax.ShapeDtypeStruct((N, H), f32),
        )(h2, agg2, eps.reshape(1, 1), W1, W1,
          b1.reshape(1, H), g1.reshape(1, H), be1.reshape(1, H))
        h1 = pl.pallas_call(
            _mm_split_body,
            grid=(N // RB,),
            in_specs=[
                pl.BlockSpec((RB, H), lambda r: (r, 0)),
                pl.BlockSpec((H, H), lambda r: (0, 0)),
                pl.BlockSpec((1, H), lambda r: (0, 0)),
            ],
            out_specs=pl.BlockSpec((2, RB, HH), lambda r: (0, r, 0)),
            out_shape=jax.ShapeDtypeStruct((2, N, HH), f32),
        )(z, W2, b2.reshape(1, H)).reshape(2 * N, HH)
        u = mm_bn(h1, mW1, mb1, mg, mbe)
        h2 = pl.pallas_call(
            functools.partial(_mm_res_body, use_res=(i > 0)),
            grid=(N // RB,),
            in_specs=[
                pl.BlockSpec((RB, H), lambda r: (r, 0)),
                pl.BlockSpec((2, RB, HH), lambda r: (0, r, 0)),
                pl.BlockSpec((H, H), lambda r: (0, 0)),
                pl.BlockSpec((1, H), lambda r: (0, 0)),
            ],
            out_specs=pl.BlockSpec((2, RB, HH), lambda r: (0, r, 0)),
            out_shape=jax.ShapeDtypeStruct((2, N, HH), f32),
        )(u, h2.reshape(2, N, HH), mW2, mb2.reshape(1, H)).reshape(2 * N, HH)

    out = pl.pallas_call(
        _pool_body,
        out_shape=jax.ShapeDtypeStruct((1, 2 * H), f32),
    )(h2, W_pool, b_pool.reshape(1, H))
    return out


# speedup vs baseline: 5.0304x; 1.3551x over previous
import functools

import jax
import jax.numpy as jnp
from jax import lax
from jax.experimental import pallas as pl
from jax.experimental.pallas import tpu as pltpu
from jax.experimental.pallas import tpu_sc as plsc

N = 10000
E = 320000
H = 256
HH = 128

NT = 16
K = 80
EPT = E // NT
NCH = EPT // K
STRIPE = 640
LAST = N - 15 * STRIPE

_PREC = jax.lax.Precision.DEFAULT


def _dot(a, b):
    return jax.lax.dot(a, b, precision=_PREC, preferred_element_type=jnp.float32)


def _bn_relu(t, g, b):
    m = jnp.mean(t, axis=0, keepdims=True)
    v = jnp.mean(t * t, axis=0, keepdims=True) - m * m
    t = (t - m) * jax.lax.rsqrt(v + 1e-5) * g + b
    return jnp.maximum(t, 0.0)


def _h_in_body(x_ref, w_ref, b_ref, o_ref):
    h = _dot(x_ref[...], w_ref[...]) + b_ref[...]
    o_ref[0] = h[:, :HH]
    o_ref[1] = h[:, HH:]


def _edge_body(ea_ref, w_ref, b_ref, o_ref):
    e = _dot(ea_ref[...], w_ref[...]) + b_ref[...]
    o_ref[0] = e[:, :HH]
    o_ref[1] = e[:, HH:]


def _conv1_body(h_ref, agg_ref, eps_ref, wa_ref, wb_ref, b_ref, g_ref,
                be_ref, o_ref):
    sc = 1.0 + eps_ref[0, 0]
    harr = h_ref[...]
    aarr = agg_ref[...]
    t0 = sc * harr[:N] + aarr[:N]
    t1 = sc * harr[N:] + aarr[N:]
    z = _dot(t0, wa_ref[...]) + _dot(t1, wb_ref[...]) + b_ref[...]
    o_ref[...] = _bn_relu(z, g_ref[...], be_ref[...])


def _mm_bn_body(t_ref, wa_ref, wb_ref, b_ref, g_ref, be_ref, o_ref):
    tarr = t_ref[...]
    z = _dot(tarr[:N], wa_ref[...]) + _dot(tarr[N:], wb_ref[...]) + b_ref[...]
    o_ref[...] = _bn_relu(z, g_ref[...], be_ref[...])


def _mm_split_body(z_ref, w_ref, b_ref, o_ref):
    zz = _dot(z_ref[...], w_ref[...]) + b_ref[...]
    o_ref[0] = zz[:, :HH]
    o_ref[1] = zz[:, HH:]


def _mm_res_body(z_ref, res_ref, w_ref, b_ref, o_ref, *, use_res):
    zz = _dot(z_ref[...], w_ref[...]) + b_ref[...]
    u0 = zz[:, :HH]
    u1 = zz[:, HH:]
    if use_res:
        u0 = u0 + res_ref[0]
        u1 = u1 + res_ref[1]
    o_ref[0] = jnp.where(u0 >= 0, u0, 0.1 * u0)
    o_ref[1] = jnp.where(u1 >= 0, u1, 0.1 * u1)


def _pool_body(h_ref, w_ref, b_ref, o_ref):
    harr = h_ref[...]
    h = jnp.concatenate([harr[:N], harr[N:]], axis=-1)
    hp = _dot(h, w_ref[...]) + b_ref[...]
    mx = jnp.max(hp, axis=0, keepdims=True)
    mn = jnp.mean(hp, axis=0, keepdims=True)
    o_ref[...] = jnp.concatenate([mx, mn], axis=-1)


def _sc_agg(h2, emb2, row, col2):
    mesh = plsc.VectorSubcoreMesh(core_axis_name="c", subcore_axis_name="s")

    @functools.partial(
        pl.kernel,
        out_type=jax.ShapeDtypeStruct((2 * N, HH), jnp.float32),
        mesh=mesh,
        scratch_types=[
            pltpu.VMEM((2, K), jnp.int32),
            pltpu.VMEM((2, K), jnp.int32),
            pltpu.VMEM((2, K, HH), jnp.float32),
            pltpu.VMEM((2, K, HH), jnp.float32),
            pltpu.VMEM_SHARED((N, HH), jnp.float32),
            pltpu.SemaphoreType.DMA,
            pltpu.SemaphoreType.DMA,
            pltpu.SemaphoreType.DMA,
            pltpu.SemaphoreType.DMA,
            pltpu.SemaphoreType.DMA,
            pltpu.SemaphoreType.DMA,
            pltpu.SemaphoreType.DMA,
            pltpu.SemaphoreType.DMA,
            pltpu.SemaphoreType.DMA,
            pltpu.SemaphoreType.DMA,
        ],
    )
    def agg_kernel(h_hbm, emb_hbm, row_hbm, col_hbm, out_hbm,
                   colv, rowv, hr, em, acc,
                   sg0, sg1, se0, se1, ss0, ss1, sic0, sic1, sir0, sir1):
        c = lax.axis_index("c")
        s = lax.axis_index("s")
        sg = (sg0, sg1)
        se = (se0, se1)
        ss = (ss0, ss1)
        sic = (sic0, sic1)
        sir = (sir0, sir1)
        ioff = s * EPT
        eoff = c * E + s * EPT

        @pl.loop(0, K)
        def _(j):
            for i in range(HH // 16):
                hr[0, j, pl.ds(i * 16, 16)] = jnp.zeros((16,), jnp.float32)

        base_r = s * STRIPE

        @pl.when(s < 15)
        def _():
            for t in range(STRIPE // K):
                pltpu.async_copy(hr.at[0], acc.at[pl.ds(base_r + t * K, K)],
                                 sg0)

        @pl.when(s == 15)
        def _():
            for t in range(LAST // K):
                pltpu.async_copy(hr.at[0], acc.at[pl.ds(base_r + t * K, K)],
                                 sg0)

        def col_fetch(ci, p):
            pltpu.async_copy(col_hbm.at[pl.ds(c * E + ioff + ci * K, K)],
                             colv.at[p], sic[p])

        def col_wait(p):
            pltpu.make_async_copy(col_hbm.at[pl.ds(ioff, K)],
                                  colv.at[p], sic[p]).wait()

        def row_fetch(ci, p):
            pltpu.async_copy(row_hbm.at[pl.ds(ioff + ci * K, K)],
                             rowv.at[p], sir[p])

        def row_wait(p):
            pltpu.make_async_copy(row_hbm.at[pl.ds(ioff, K)],
                                  rowv.at[p], sir[p]).wait()

        def g_fetch(p):
            pltpu.async_copy(h_hbm.at[colv.at[p]], hr.at[p], sg[p])

        def g_wait(p):
            pltpu.make_async_copy(h_hbm.at[colv.at[p]], hr.at[p],
                                  sg[p]).wait()

        def e_fetch(ci, p):
            pltpu.async_copy(emb_hbm.at[pl.ds(eoff + ci * K, K)],
                             em.at[p], se[p])

        def e_wait(ci, p):
            pltpu.make_async_copy(emb_hbm.at[pl.ds(eoff + ci * K, K)],
                                  em.at[p], se[p]).wait()

        def sc_drain(p):
            pltpu.make_async_copy(hr.at[p], acc.at[rowv.at[p]],
                                  ss[p]).wait()

        col_fetch(0, 0)
        col_fetch(1, 1)
        row_fetch(0, 0)
        e_fetch(0, 0)
        e_fetch(1, 1)

        @pl.when(s < 15)
        def _():
            for t in range(STRIPE // K):
                pltpu.make_async_copy(
                    hr.at[0], acc.at[pl.ds(base_r + t * K, K)], sg0).wait()

        @pl.when(s == 15)
        def _():
            for t in range(LAST // K):
                pltpu.make_async_copy(
                    hr.at[0], acc.at[pl.ds(base_r + t * K, K)], sg0).wait()

        col_wait(0)
        g_fetch(0)
        plsc.subcore_barrier()

        @pl.loop(0, NCH // 2)
        def _(cj):
            ci0 = cj * 2
            for p in range(2):
                ci = ci0 + p
                p1 = (p + 1) % 2

                g_wait(p)
                e_wait(ci, p)

                @pl.when(ci >= 1)
                def _():
                    sc_drain(p1)

                @pl.when(ci + 1 < NCH)
                def _():
                    row_fetch(ci + 1, p1)

                @pl.when(ci + 2 < NCH)
                def _():
                    col_fetch(ci + 2, p)

                @pl.when(ci + 1 < NCH)
                def _():
                    col_wait(p1)
                    g_fetch(p1)

                @plsc.parallel_loop(0, K, unroll=4)
                def _(j):
                    for i in range(HH // 16):
                        sl = pl.ds(i * 16, 16)
                        hr[p, j, sl] = hr[p, j, sl] * em[p, j, sl]

                row_wait(p)
                pltpu.async_copy(hr.at[p], acc.at[rowv.at[p]], ss[p],
                                 add=True)

                @pl.when(ci + 2 < NCH)
                def _():
                    e_fetch(ci + 2, p)

        sc_drain(1)

        plsc.subcore_barrier()

        @pl.when(s < 15)
        def _():
            pltpu.sync_copy(acc.at[pl.ds(base_r, STRIPE)],
                            out_hbm.at[pl.ds(c * N + base_r, STRIPE)])

        @pl.when(s == 15)
        def _():
            pltpu.sync_copy(acc.at[pl.ds(base_r, LAST)],
                            out_hbm.at[pl.ds(c * N + base_r, LAST)])

    return agg_kernel(h2, emb2, row, col2)


def kernel(x, edge_index, edge_attr,
           W_in, b_in, W_edge, b_edge,
           c0_eps, c0_W1, c0_b1, c0_g1, c0_be1, c0_W2, c0_b2,
           m0_W1, m0_b1, m0_g, m0_be, m0_W2, m0_b2,
           c1_eps, c1_W1, c1_b1, c1_g1, c1_be1, c1_W2, c1_b2,
           m1_W1, m1_b1, m1_g, m1_be, m1_W2, m1_b2,
           c2_eps, c2_W1, c2_b1, c2_g1, c2_be1, c2_W2, c2_b2,
           m2_W1, m2_b1, m2_g, m2_be, m2_W2, m2_b2,
           W_pool, b_pool):
    f32 = jnp.float32
    row = edge_index[0]
    col = edge_index[1]
    col2 = jnp.concatenate([col, col + N])

    h2 = pl.pallas_call(
        _h_in_body,
        out_shape=jax.ShapeDtypeStruct((2, N, HH), f32),
    )(x, W_in, b_in.reshape(1, H)).reshape(2 * N, HH)

    EB = 8000
    emb2 = pl.pallas_call(
        _edge_body,
        grid=(E // EB,),
        in_specs=[
            pl.BlockSpec((EB, 9), lambda i: (i, 0)),
            pl.BlockSpec((9, H), lambda i: (0, 0)),
            pl.BlockSpec((1, H), lambda i: (0, 0)),
        ],
        out_specs=pl.BlockSpec((2, EB, HH), lambda i: (0, i, 0)),
        out_shape=jax.ShapeDtypeStruct((2, E, HH), f32),
    )(edge_attr, W_edge, b_edge.reshape(1, H)).reshape(2 * E, HH)

    layers = [
        (c0_eps, c0_W1, c0_b1, c0_g1, c0_be1, c0_W2, c0_b2,
         m0_W1, m0_b1, m0_g, m0_be, m0_W2, m0_b2),
        (c1_eps, c1_W1, c1_b1, c1_g1, c1_be1, c1_W2, c1_b2,
         m1_W1, m1_b1, m1_g, m1_be, m1_W2, m1_b2),
        (c2_eps, c2_W1, c2_b1, c2_g1, c2_be1, c2_W2, c2_b2,
         m2_W1, m2_b1, m2_g, m2_be, m2_W2, m2_b2),
    ]

    RB = 2000

    def mm_bn(t2, W, b, g, be):
        return pl.pallas_call(
            _mm_bn_body,
            grid=(H // HH,),
            in_specs=[
                pl.BlockSpec((2 * N, HH), lambda j: (0, 0)),
                pl.BlockSpec((HH, HH), lambda j: (0, j)),
                pl.BlockSpec((HH, HH), lambda j: (1, j)),
                pl.BlockSpec((1, HH), lambda j: (0, j)),
                pl.BlockSpec((1, HH), lambda j: (0, j)),
                pl.BlockSpec((1, HH), lambda j: (0, j)),
            ],
            out_specs=pl.BlockSpec((N, HH), lambda j: (0, j)),
            out_shape=jax.ShapeDtypeStruct((N, H), f32),
        )(t2, W, W, b.reshape(1, H), g.reshape(1, H), be.reshape(1, H))

    for i, (eps, W1, b1, g1, be1, W2, b2,
            mW1, mb1, mg, mbe, mW2, mb2) in enumerate(layers):
        agg2 = _sc_agg(h2, emb2, row, col2)
        z = pl.pallas_call(
            _conv1_body,
            grid=(H // HH,),
            in_specs=[
                pl.BlockSpec((2 * N, HH), lambda j: (0, 0)),
                pl.BlockSpec((2 * N, HH), lambda j: (0, 0)),
                pl.BlockSpec((1, 1), lambda j: (0, 0)),
                pl.BlockSpec((HH, HH), lambda j: (0, j)),
                pl.BlockSpec((HH, HH), lambda j: (1, j)),
                pl.BlockSpec((1, HH), lambda j: (0, j)),
                pl.BlockSpec((1, HH), lambda j: (0, j)),
                pl.BlockSpec((1, HH), lambda j: (0, j)),
            ],
            out_specs=pl.BlockSpec((N, HH), lambda j: (0, j)),
            out_shape=jax.ShapeDtypeStruct((N, H), f32),
        )(h2, agg2, eps.reshape(1, 1), W1, W1,
          b1.reshape(1, H), g1.reshape(1, H), be1.reshape(1, H))
        h1 = pl.pallas_call(
            _mm_split_body,
            grid=(N // RB,),
            in_specs=[
                pl.BlockSpec((RB, H), lambda r: (r, 0)),
                pl.BlockSpec((H, H), lambda r: (0, 0)),
                pl.BlockSpec((1, H), lambda r: (0, 0)),
            ],
            out_specs=pl.BlockSpec((2, RB, HH), lambda r: (0, r, 0)),
            out_shape=jax.ShapeDtypeStruct((2, N, HH), f32),
        )(z, W2, b2.reshape(1, H)).reshape(2 * N, HH)
        u = mm_bn(h1, mW1, mb1, mg, mbe)
        h2 = pl.pallas_call(
            functools.partial(_mm_res_body, use_res=(i > 0)),
            grid=(N // RB,),
            in_specs=[
                pl.BlockSpec((RB, H), lambda r: (r, 0)),
                pl.BlockSpec((2, RB, HH), lambda r: (0, r, 0)),
                pl.BlockSpec((H, H), lambda r: (0, 0)),
                pl.BlockSpec((1, H), lambda r: (0, 0)),
            ],
            out_specs=pl.BlockSpec((2, RB, HH), lambda r: (0, r, 0)),
            out_shape=jax.ShapeDtypeStruct((2, N, HH), f32),
        )(u, h2.reshape(2, N, HH), mW2, mb2.reshape(1, H)).reshape(2 * N, HH)

    out = pl.pallas_call(
        _pool_body,
        out_shape=jax.ShapeDtypeStruct((1, 2 * H), f32),
    )(h2, W_pool, b_pool.reshape(1, H))
    return out


# speedup vs baseline: 5.1102x; 1.0159x over previous
import functools

import jax
import jax.numpy as jnp
from jax import lax
from jax.experimental import pallas as pl
from jax.experimental.pallas import tpu as pltpu
from jax.experimental.pallas import tpu_sc as plsc

N = 10000
E = 320000
H = 256
HH = 128

NT = 16
K = 80
EPT = E // NT
NCH = EPT // K
STRIPE = 640
LAST = N - 15 * STRIPE

_PREC = jax.lax.Precision.DEFAULT


def _dot(a, b):
    return jax.lax.dot(a, b, precision=_PREC, preferred_element_type=jnp.float32)


def _bn_relu(t, g, b):
    m = jnp.mean(t, axis=0, keepdims=True)
    v = jnp.mean(t * t, axis=0, keepdims=True) - m * m
    t = (t - m) * jax.lax.rsqrt(v + 1e-5) * g + b
    return jnp.maximum(t, 0.0)


def _h_in_body(x_ref, w_ref, b_ref, o_ref):
    h = _dot(x_ref[...], w_ref[...]) + b_ref[...]
    o_ref[0] = h[:, :HH]
    o_ref[1] = h[:, HH:]


def _edge_body(ea_ref, w_ref, b_ref, o_ref):
    e = _dot(ea_ref[...], w_ref[...]) + b_ref[...]
    o_ref[0] = e[:, :HH]
    o_ref[1] = e[:, HH:]


def _conv1_body(h_ref, agg_ref, eps_ref, wa_ref, wb_ref, b_ref, g_ref,
                be_ref, o_ref):
    sc = 1.0 + eps_ref[0, 0]
    harr = h_ref[...]
    aarr = agg_ref[...]
    t0 = sc * harr[:N] + aarr[:N]
    t1 = sc * harr[N:] + aarr[N:]
    z = _dot(t0, wa_ref[...]) + _dot(t1, wb_ref[...]) + b_ref[...]
    o_ref[...] = _bn_relu(z, g_ref[...], be_ref[...])


def _mm2_bn_body(z_ref, w2_ref, b2_ref, w_ref, b_ref, g_ref, be_ref, o_ref):
    h1 = _dot(z_ref[...], w2_ref[...]) + b2_ref[...]
    u = _dot(h1, w_ref[...]) + b_ref[...]
    o_ref[...] = _bn_relu(u, g_ref[...], be_ref[...])


def _mm_res_body(z_ref, res_ref, w_ref, b_ref, o_ref, *, use_res):
    zz = _dot(z_ref[...], w_ref[...]) + b_ref[...]
    u0 = zz[:, :HH]
    u1 = zz[:, HH:]
    if use_res:
        u0 = u0 + res_ref[0]
        u1 = u1 + res_ref[1]
    o_ref[0] = jnp.where(u0 >= 0, u0, 0.1 * u0)
    o_ref[1] = jnp.where(u1 >= 0, u1, 0.1 * u1)


def _pool_body(h_ref, w_ref, b_ref, o_ref):
    harr = h_ref[...]
    h = jnp.concatenate([harr[:N], harr[N:]], axis=-1)
    hp = _dot(h, w_ref[...]) + b_ref[...]
    mx = jnp.max(hp, axis=0, keepdims=True)
    mn = jnp.mean(hp, axis=0, keepdims=True)
    o_ref[...] = jnp.concatenate([mx, mn], axis=-1)


def _sc_agg(h2, emb2, row, col2):
    mesh = plsc.VectorSubcoreMesh(core_axis_name="c", subcore_axis_name="s")

    @functools.partial(
        pl.kernel,
        out_type=jax.ShapeDtypeStruct((2 * N, HH), jnp.float32),
        mesh=mesh,
        scratch_types=[
            pltpu.VMEM((2, K), jnp.int32),
            pltpu.VMEM((2, K), jnp.int32),
            pltpu.VMEM((2, K, HH), jnp.float32),
            pltpu.VMEM((2, K, HH), jnp.float32),
            pltpu.VMEM_SHARED((N, HH), jnp.float32),
            pltpu.SemaphoreType.DMA,
            pltpu.SemaphoreType.DMA,
            pltpu.SemaphoreType.DMA,
            pltpu.SemaphoreType.DMA,
            pltpu.SemaphoreType.DMA,
            pltpu.SemaphoreType.DMA,
            pltpu.SemaphoreType.DMA,
            pltpu.SemaphoreType.DMA,
            pltpu.SemaphoreType.DMA,
            pltpu.SemaphoreType.DMA,
        ],
    )
    def agg_kernel(h_hbm, emb_hbm, row_hbm, col_hbm, out_hbm,
                   colv, rowv, hr, em, acc,
                   sg0, sg1, se0, se1, ss0, ss1, sic0, sic1, sir0, sir1):
        c = lax.axis_index("c")
        s = lax.axis_index("s")
        sg = (sg0, sg1)
        se = (se0, se1)
        ss = (ss0, ss1)
        sic = (sic0, sic1)
        sir = (sir0, sir1)
        ioff = s * EPT
        eoff = c * E + s * EPT

        @pl.loop(0, K)
        def _(j):
            for i in range(HH // 16):
                hr[0, j, pl.ds(i * 16, 16)] = jnp.zeros((16,), jnp.float32)

        base_r = s * STRIPE

        @pl.when(s < 15)
        def _():
            for t in range(STRIPE // K):
                pltpu.async_copy(hr.at[0], acc.at[pl.ds(base_r + t * K, K)],
                                 sg0)

        @pl.when(s == 15)
        def _():
            for t in range(LAST // K):
                pltpu.async_copy(hr.at[0], acc.at[pl.ds(base_r + t * K, K)],
                                 sg0)

        def col_fetch(ci, p):
            pltpu.async_copy(col_hbm.at[pl.ds(c * E + ioff + ci * K, K)],
                             colv.at[p], sic[p])

        def col_wait(p):
            pltpu.make_async_copy(col_hbm.at[pl.ds(ioff, K)],
                                  colv.at[p], sic[p]).wait()

        def row_fetch(ci, p):
            pltpu.async_copy(row_hbm.at[pl.ds(ioff + ci * K, K)],
                             rowv.at[p], sir[p])

        def row_wait(p):
            pltpu.make_async_copy(row_hbm.at[pl.ds(ioff, K)],
                                  rowv.at[p], sir[p]).wait()

        def g_fetch(p):
            pltpu.async_copy(h_hbm.at[colv.at[p]], hr.at[p], sg[p])

        def g_wait(p):
            pltpu.make_async_copy(h_hbm.at[colv.at[p]], hr.at[p],
                                  sg[p]).wait()

        def e_fetch(ci, p):
            pltpu.async_copy(emb_hbm.at[pl.ds(eoff + ci * K, K)],
                             em.at[p], se[p])

        def e_wait(ci, p):
            pltpu.make_async_copy(emb_hbm.at[pl.ds(eoff + ci * K, K)],
                                  em.at[p], se[p]).wait()

        def sc_drain(p):
            pltpu.make_async_copy(hr.at[p], acc.at[rowv.at[p]],
                                  ss[p]).wait()

        col_fetch(0, 0)
        col_fetch(1, 1)
        row_fetch(0, 0)
        e_fetch(0, 0)
        e_fetch(1, 1)

        @pl.when(s < 15)
        def _():
            for t in range(STRIPE // K):
                pltpu.make_async_copy(
                    hr.at[0], acc.at[pl.ds(base_r + t * K, K)], sg0).wait()

        @pl.when(s == 15)
        def _():
            for t in range(LAST // K):
                pltpu.make_async_copy(
                    hr.at[0], acc.at[pl.ds(base_r + t * K, K)], sg0).wait()

        col_wait(0)
        g_fetch(0)
        plsc.subcore_barrier()

        @pl.loop(0, NCH // 2)
        def _(cj):
            ci0 = cj * 2
            for p in range(2):
                ci = ci0 + p
                p1 = (p + 1) % 2

                g_wait(p)
                e_wait(ci, p)

                @pl.when(ci >= 1)
                def _():
                    sc_drain(p1)

                @pl.when(ci + 1 < NCH)
                def _():
                    row_fetch(ci + 1, p1)

                @pl.when(ci + 2 < NCH)
                def _():
                    col_fetch(ci + 2, p)

                @pl.when(ci + 1 < NCH)
                def _():
                    col_wait(p1)
                    g_fetch(p1)

                @plsc.parallel_loop(0, K, unroll=4)
                def _(j):
                    for i in range(HH // 16):
                        sl = pl.ds(i * 16, 16)
                        hr[p, j, sl] = hr[p, j, sl] * em[p, j, sl]

                row_wait(p)
                pltpu.async_copy(hr.at[p], acc.at[rowv.at[p]], ss[p],
                                 add=True)

                @pl.when(ci + 2 < NCH)
                def _():
                    e_fetch(ci + 2, p)

        sc_drain(1)

        plsc.subcore_barrier()

        @pl.when(s < 15)
        def _():
            pltpu.sync_copy(acc.at[pl.ds(base_r, STRIPE)],
                            out_hbm.at[pl.ds(c * N + base_r, STRIPE)])

        @pl.when(s == 15)
        def _():
            pltpu.sync_copy(acc.at[pl.ds(base_r, LAST)],
                            out_hbm.at[pl.ds(c * N + base_r, LAST)])

    return agg_kernel(h2, emb2, row, col2)


def kernel(x, edge_index, edge_attr,
           W_in, b_in, W_edge, b_edge,
           c0_eps, c0_W1, c0_b1, c0_g1, c0_be1, c0_W2, c0_b2,
           m0_W1, m0_b1, m0_g, m0_be, m0_W2, m0_b2,
           c1_eps, c1_W1, c1_b1, c1_g1, c1_be1, c1_W2, c1_b2,
           m1_W1, m1_b1, m1_g, m1_be, m1_W2, m1_b2,
           c2_eps, c2_W1, c2_b1, c2_g1, c2_be1, c2_W2, c2_b2,
           m2_W1, m2_b1, m2_g, m2_be, m2_W2, m2_b2,
           W_pool, b_pool):
    f32 = jnp.float32
    row = edge_index[0]
    col = edge_index[1]
    col2 = jnp.concatenate([col, col + N])

    h2 = pl.pallas_call(
        _h_in_body,
        out_shape=jax.ShapeDtypeStruct((2, N, HH), f32),
    )(x, W_in, b_in.reshape(1, H)).reshape(2 * N, HH)

    EB = 8000
    emb2 = pl.pallas_call(
        _edge_body,
        grid=(E // EB,),
        in_specs=[
            pl.BlockSpec((EB, 9), lambda i: (i, 0)),
            pl.BlockSpec((9, H), lambda i: (0, 0)),
            pl.BlockSpec((1, H), lambda i: (0, 0)),
        ],
        out_specs=pl.BlockSpec((2, EB, HH), lambda i: (0, i, 0)),
        out_shape=jax.ShapeDtypeStruct((2, E, HH), f32),
    )(edge_attr, W_edge, b_edge.reshape(1, H)).reshape(2 * E, HH)

    layers = [
        (c0_eps, c0_W1, c0_b1, c0_g1, c0_be1, c0_W2, c0_b2,
         m0_W1, m0_b1, m0_g, m0_be, m0_W2, m0_b2),
        (c1_eps, c1_W1, c1_b1, c1_g1, c1_be1, c1_W2, c1_b2,
         m1_W1, m1_b1, m1_g, m1_be, m1_W2, m1_b2),
        (c2_eps, c2_W1, c2_b1, c2_g1, c2_be1, c2_W2, c2_b2,
         m2_W1, m2_b1, m2_g, m2_be, m2_W2, m2_b2),
    ]

    RB = 2000

    for i, (eps, W1, b1, g1, be1, W2, b2,
            mW1, mb1, mg, mbe, mW2, mb2) in enumerate(layers):
        agg2 = _sc_agg(h2, emb2, row, col2)
        z = pl.pallas_call(
            _conv1_body,
            grid=(H // HH,),
            in_specs=[
                pl.BlockSpec((2 * N, HH), lambda j: (0, 0)),
                pl.BlockSpec((2 * N, HH), lambda j: (0, 0)),
                pl.BlockSpec((1, 1), lambda j: (0, 0)),
                pl.BlockSpec((HH, HH), lambda j: (0, j)),
                pl.BlockSpec((HH, HH), lambda j: (1, j)),
                pl.BlockSpec((1, HH), lambda j: (0, j)),
                pl.BlockSpec((1, HH), lambda j: (0, j)),
                pl.BlockSpec((1, HH), lambda j: (0, j)),
            ],
            out_specs=pl.BlockSpec((N, HH), lambda j: (0, j)),
            out_shape=jax.ShapeDtypeStruct((N, H), f32),
        )(h2, agg2, eps.reshape(1, 1), W1, W1,
          b1.reshape(1, H), g1.reshape(1, H), be1.reshape(1, H))
        u = pl.pallas_call(
            _mm2_bn_body,
            grid=(H // HH,),
            in_specs=[
                pl.BlockSpec((N, H), lambda j: (0, 0)),
                pl.BlockSpec((H, H), lambda j: (0, 0)),
                pl.BlockSpec((1, H), lambda j: (0, 0)),
                pl.BlockSpec((H, HH), lambda j: (0, j)),
                pl.BlockSpec((1, HH), lambda j: (0, j)),
                pl.BlockSpec((1, HH), lambda j: (0, j)),
                pl.BlockSpec((1, HH), lambda j: (0, j)),
            ],
            out_specs=pl.BlockSpec((N, HH), lambda j: (0, j)),
            out_shape=jax.ShapeDtypeStruct((N, H), f32),
        )(z, W2, b2.reshape(1, H), mW1,
          mb1.reshape(1, H), mg.reshape(1, H), mbe.reshape(1, H))
        h2 = pl.pallas_call(
            functools.partial(_mm_res_body, use_res=(i > 0)),
            grid=(N // RB,),
            in_specs=[
                pl.BlockSpec((RB, H), lambda r: (r, 0)),
                pl.BlockSpec((2, RB, HH), lambda r: (0, r, 0)),
                pl.BlockSpec((H, H), lambda r: (0, 0)),
                pl.BlockSpec((1, H), lambda r: (0, 0)),
            ],
            out_specs=pl.BlockSpec((2, RB, HH), lambda r: (0, r, 0)),
            out_shape=jax.ShapeDtypeStruct((2, N, HH), f32),
        )(u, h2.reshape(2, N, HH), mW2, mb2.reshape(1, H)).reshape(2 * N, HH)

    out = pl.pallas_call(
        _pool_body,
        out_shape=jax.ShapeDtypeStruct((1, 2 * H), f32),
    )(h2, W_pool, b_pool.reshape(1, H))
    return out


# speedup vs baseline: 5.1153x; 1.0010x over previous
import functools

import jax
import jax.numpy as jnp
from jax import lax
from jax.experimental import pallas as pl
from jax.experimental.pallas import tpu as pltpu
from jax.experimental.pallas import tpu_sc as plsc

N = 10000
E = 320000
H = 256
HH = 128

NT = 16
K = 80
EPT = E // NT
NCH = EPT // K
STRIPE = 640
LAST = N - 15 * STRIPE

_PREC = jax.lax.Precision.DEFAULT


def _dot(a, b):
    return jax.lax.dot(a, b, precision=_PREC, preferred_element_type=jnp.float32)


def _bn_relu(t, g, b):
    m = jnp.mean(t, axis=0, keepdims=True)
    v = jnp.mean(t * t, axis=0, keepdims=True) - m * m
    t = (t - m) * jax.lax.rsqrt(v + 1e-5) * g + b
    return jnp.maximum(t, 0.0)


def _h_in_body(x_ref, w_ref, b_ref, o_ref):
    h = _dot(x_ref[...], w_ref[...]) + b_ref[...]
    o_ref[0] = h[:, :HH]
    o_ref[1] = h[:, HH:]


def _edge_body(ea_ref, w_ref, b_ref, o_ref):
    e = _dot(ea_ref[...], w_ref[...]) + b_ref[...]
    o_ref[0] = e[:, :HH]
    o_ref[1] = e[:, HH:]


def _conv1_body(h_ref, agg_ref, eps_ref, wa_ref, wb_ref, b_ref, g_ref,
                be_ref, o_ref):
    sc = 1.0 + eps_ref[0, 0]
    harr = h_ref[...]
    aarr = agg_ref[...]
    t0 = sc * harr[:N] + aarr[:N]
    t1 = sc * harr[N:] + aarr[N:]
    z = _dot(t0, wa_ref[...]) + _dot(t1, wb_ref[...]) + b_ref[...]
    o_ref[...] = _bn_relu(z, g_ref[...], be_ref[...])


def _mm2_bn_body(z_ref, w2_ref, b2_ref, w_ref, b_ref, g_ref, be_ref, o_ref):
    h1 = _dot(z_ref[...], w2_ref[...]) + b2_ref[...]
    u = _dot(h1, w_ref[...]) + b_ref[...]
    o_ref[...] = _bn_relu(u, g_ref[...], be_ref[...])


def _mm_res_body(z_ref, res_ref, w_ref, b_ref, o_ref, *, use_res):
    zz = _dot(z_ref[...], w_ref[...]) + b_ref[...]
    u0 = zz[:, :HH]
    u1 = zz[:, HH:]
    if use_res:
        u0 = u0 + res_ref[0]
        u1 = u1 + res_ref[1]
    o_ref[0] = jnp.where(u0 >= 0, u0, 0.1 * u0)
    o_ref[1] = jnp.where(u1 >= 0, u1, 0.1 * u1)


def _mm_res_pool_body(z_ref, res_ref, w_ref, b_ref, wp_ref, bp_ref,
                      o_ref, o2_ref):
    zz = _dot(z_ref[...], w_ref[...]) + b_ref[...]
    u0 = zz[:, :HH] + res_ref[0]
    u1 = zz[:, HH:] + res_ref[1]
    h0 = jnp.where(u0 >= 0, u0, 0.1 * u0)
    h1 = jnp.where(u1 >= 0, u1, 0.1 * u1)
    o_ref[0] = h0
    o_ref[1] = h1
    wp = wp_ref[...]
    hp = _dot(h0, wp[:HH]) + _dot(h1, wp[HH:]) + bp_ref[...]
    mx = jnp.max(hp, axis=0, keepdims=True)
    mn = jnp.sum(hp, axis=0, keepdims=True) * (1.0 / N)
    r = pl.program_id(0)

    @pl.when(r == 0)
    def _():
        o2_ref[...] = jnp.concatenate([mx, mn], axis=-1)

    @pl.when(r > 0)
    def _():
        prev = o2_ref[...]
        o2_ref[...] = jnp.concatenate(
            [jnp.maximum(prev[:, :H], mx), prev[:, H:] + mn], axis=-1)


def _sc_agg(h2, emb2, row, col2):
    mesh = plsc.VectorSubcoreMesh(core_axis_name="c", subcore_axis_name="s")

    @functools.partial(
        pl.kernel,
        out_type=jax.ShapeDtypeStruct((2 * N, HH), jnp.float32),
        mesh=mesh,
        scratch_types=[
            pltpu.VMEM((2, K), jnp.int32),
            pltpu.VMEM((2, K), jnp.int32),
            pltpu.VMEM((2, K, HH), jnp.float32),
            pltpu.VMEM((2, K, HH), jnp.float32),
            pltpu.VMEM_SHARED((N, HH), jnp.float32),
            pltpu.SemaphoreType.DMA,
            pltpu.SemaphoreType.DMA,
            pltpu.SemaphoreType.DMA,
            pltpu.SemaphoreType.DMA,
            pltpu.SemaphoreType.DMA,
            pltpu.SemaphoreType.DMA,
            pltpu.SemaphoreType.DMA,
            pltpu.SemaphoreType.DMA,
            pltpu.SemaphoreType.DMA,
            pltpu.SemaphoreType.DMA,
        ],
    )
    def agg_kernel(h_hbm, emb_hbm, row_hbm, col_hbm, out_hbm,
                   colv, rowv, hr, em, acc,
                   sg0, sg1, se0, se1, ss0, ss1, sic0, sic1, sir0, sir1):
        c = lax.axis_index("c")
        s = lax.axis_index("s")
        sg = (sg0, sg1)
        se = (se0, se1)
        ss = (ss0, ss1)
        sic = (sic0, sic1)
        sir = (sir0, sir1)
        ioff = s * EPT
        eoff = c * E + s * EPT

        @pl.loop(0, K)
        def _(j):
            for i in range(HH // 16):
                hr[0, j, pl.ds(i * 16, 16)] = jnp.zeros((16,), jnp.float32)

        base_r = s * STRIPE

        @pl.when(s < 15)
        def _():
            for t in range(STRIPE // K):
                pltpu.async_copy(hr.at[0], acc.at[pl.ds(base_r + t * K, K)],
                                 sg0)

        @pl.when(s == 15)
        def _():
            for t in range(LAST // K):
                pltpu.async_copy(hr.at[0], acc.at[pl.ds(base_r + t * K, K)],
                                 sg0)

        def col_fetch(ci, p):
            pltpu.async_copy(col_hbm.at[pl.ds(c * E + ioff + ci * K, K)],
                             colv.at[p], sic[p])

        def col_wait(p):
            pltpu.make_async_copy(col_hbm.at[pl.ds(ioff, K)],
                                  colv.at[p], sic[p]).wait()

        def row_fetch(ci, p):
            pltpu.async_copy(row_hbm.at[pl.ds(ioff + ci * K, K)],
                             rowv.at[p], sir[p])

        def row_wait(p):
            pltpu.make_async_copy(row_hbm.at[pl.ds(ioff, K)],
                                  rowv.at[p], sir[p]).wait()

        def g_fetch(p):
            pltpu.async_copy(h_hbm.at[colv.at[p]], hr.at[p], sg[p])

        def g_wait(p):
            pltpu.make_async_copy(h_hbm.at[colv.at[p]], hr.at[p],
                                  sg[p]).wait()

        def e_fetch(ci, p):
            pltpu.async_copy(emb_hbm.at[pl.ds(eoff + ci * K, K)],
                             em.at[p], se[p])

        def e_wait(ci, p):
            pltpu.make_async_copy(emb_hbm.at[pl.ds(eoff + ci * K, K)],
                                  em.at[p], se[p]).wait()

        def sc_drain(p):
            pltpu.make_async_copy(hr.at[p], acc.at[rowv.at[p]],
                                  ss[p]).wait()

        col_fetch(0, 0)
        col_fetch(1, 1)
        row_fetch(0, 0)
        e_fetch(0, 0)
        e_fetch(1, 1)

        @pl.when(s < 15)
        def _():
            for t in range(STRIPE // K):
                pltpu.make_async_copy(
                    hr.at[0], acc.at[pl.ds(base_r + t * K, K)], sg0).wait()

        @pl.when(s == 15)
        def _():
            for t in range(LAST // K):
                pltpu.make_async_copy(
                    hr.at[0], acc.at[pl.ds(base_r + t * K, K)], sg0).wait()

        col_wait(0)
        g_fetch(0)
        plsc.subcore_barrier()

        @pl.loop(0, NCH // 2)
        def _(cj):
            ci0 = cj * 2
            for p in range(2):
                ci = ci0 + p
                p1 = (p + 1) % 2

                g_wait(p)
                e_wait(ci, p)

                @pl.when(ci >= 1)
                def _():
                    sc_drain(p1)

                @pl.when(ci + 1 < NCH)
                def _():
                    row_fetch(ci + 1, p1)

                @pl.when(ci + 2 < NCH)
                def _():
                    col_fetch(ci + 2, p)

                @pl.when(ci + 1 < NCH)
                def _():
                    col_wait(p1)
                    g_fetch(p1)

                @plsc.parallel_loop(0, K, unroll=4)
                def _(j):
                    for i in range(HH // 16):
                        sl = pl.ds(i * 16, 16)
                        hr[p, j, sl] = hr[p, j, sl] * em[p, j, sl]

                row_wait(p)
                pltpu.async_copy(hr.at[p], acc.at[rowv.at[p]], ss[p],
                                 add=True)

                @pl.when(ci + 2 < NCH)
                def _():
                    e_fetch(ci + 2, p)

        sc_drain(1)

        plsc.subcore_barrier()

        @pl.when(s < 15)
        def _():
            pltpu.sync_copy(acc.at[pl.ds(base_r, STRIPE)],
                            out_hbm.at[pl.ds(c * N + base_r, STRIPE)])

        @pl.when(s == 15)
        def _():
            pltpu.sync_copy(acc.at[pl.ds(base_r, LAST)],
                            out_hbm.at[pl.ds(c * N + base_r, LAST)])

    return agg_kernel(h2, emb2, row, col2)


def kernel(x, edge_index, edge_attr,
           W_in, b_in, W_edge, b_edge,
           c0_eps, c0_W1, c0_b1, c0_g1, c0_be1, c0_W2, c0_b2,
           m0_W1, m0_b1, m0_g, m0_be, m0_W2, m0_b2,
           c1_eps, c1_W1, c1_b1, c1_g1, c1_be1, c1_W2, c1_b2,
           m1_W1, m1_b1, m1_g, m1_be, m1_W2, m1_b2,
           c2_eps, c2_W1, c2_b1, c2_g1, c2_be1, c2_W2, c2_b2,
           m2_W1, m2_b1, m2_g, m2_be, m2_W2, m2_b2,
           W_pool, b_pool):
    f32 = jnp.float32
    row = edge_index[0]
    col = edge_index[1]
    col2 = jnp.concatenate([col, col + N])

    h2 = pl.pallas_call(
        _h_in_body,
        out_shape=jax.ShapeDtypeStruct((2, N, HH), f32),
    )(x, W_in, b_in.reshape(1, H)).reshape(2 * N, HH)

    EB = 8000
    emb2 = pl.pallas_call(
        _edge_body,
        grid=(E // EB,),
        in_specs=[
            pl.BlockSpec((EB, 9), lambda i: (i, 0)),
            pl.BlockSpec((9, H), lambda i: (0, 0)),
            pl.BlockSpec((1, H), lambda i: (0, 0)),
        ],
        out_specs=pl.BlockSpec((2, EB, HH), lambda i: (0, i, 0)),
        out_shape=jax.ShapeDtypeStruct((2, E, HH), f32),
    )(edge_attr, W_edge, b_edge.reshape(1, H)).reshape(2 * E, HH)

    layers = [
        (c0_eps, c0_W1, c0_b1, c0_g1, c0_be1, c0_W2, c0_b2,
         m0_W1, m0_b1, m0_g, m0_be, m0_W2, m0_b2),
        (c1_eps, c1_W1, c1_b1, c1_g1, c1_be1, c1_W2, c1_b2,
         m1_W1, m1_b1, m1_g, m1_be, m1_W2, m1_b2),
        (c2_eps, c2_W1, c2_b1, c2_g1, c2_be1, c2_W2, c2_b2,
         m2_W1, m2_b1, m2_g, m2_be, m2_W2, m2_b2),
    ]

    RB = 2000

    for i, (eps, W1, b1, g1, be1, W2, b2,
            mW1, mb1, mg, mbe, mW2, mb2) in enumerate(layers):
        agg2 = _sc_agg(h2, emb2, row, col2)
        z = pl.pallas_call(
            _conv1_body,
            grid=(H // HH,),
            in_specs=[
                pl.BlockSpec((2 * N, HH), lambda j: (0, 0)),
                pl.BlockSpec((2 * N, HH), lambda j: (0, 0)),
                pl.BlockSpec((1, 1), lambda j: (0, 0)),
                pl.BlockSpec((HH, HH), lambda j: (0, j)),
                pl.BlockSpec((HH, HH), lambda j: (1, j)),
                pl.BlockSpec((1, HH), lambda j: (0, j)),
                pl.BlockSpec((1, HH), lambda j: (0, j)),
                pl.BlockSpec((1, HH), lambda j: (0, j)),
            ],
            out_specs=pl.BlockSpec((N, HH), lambda j: (0, j)),
            out_shape=jax.ShapeDtypeStruct((N, H), f32),
        )(h2, agg2, eps.reshape(1, 1), W1, W1,
          b1.reshape(1, H), g1.reshape(1, H), be1.reshape(1, H))
        u = pl.pallas_call(
            _mm2_bn_body,
            grid=(H // HH,),
            in_specs=[
                pl.BlockSpec((N, H), lambda j: (0, 0)),
                pl.BlockSpec((H, H), lambda j: (0, 0)),
                pl.BlockSpec((1, H), lambda j: (0, 0)),
                pl.BlockSpec((H, HH), lambda j: (0, j)),
                pl.BlockSpec((1, HH), lambda j: (0, j)),
                pl.BlockSpec((1, HH), lambda j: (0, j)),
                pl.BlockSpec((1, HH), lambda j: (0, j)),
            ],
            out_specs=pl.BlockSpec((N, HH), lambda j: (0, j)),
            out_shape=jax.ShapeDtypeStruct((N, H), f32),
        )(z, W2, b2.reshape(1, H), mW1,
          mb1.reshape(1, H), mg.reshape(1, H), mbe.reshape(1, H))
        if i < 2:
            h2 = pl.pallas_call(
                functools.partial(_mm_res_body, use_res=(i > 0)),
                grid=(N // RB,),
                in_specs=[
                    pl.BlockSpec((RB, H), lambda r: (r, 0)),
                    pl.BlockSpec((2, RB, HH), lambda r: (0, r, 0)),
                    pl.BlockSpec((H, H), lambda r: (0, 0)),
                    pl.BlockSpec((1, H), lambda r: (0, 0)),
                ],
                out_specs=pl.BlockSpec((2, RB, HH), lambda r: (0, r, 0)),
                out_shape=jax.ShapeDtypeStruct((2, N, HH), f32),
            )(u, h2.reshape(2, N, HH), mW2,
              mb2.reshape(1, H)).reshape(2 * N, HH)
        else:
            _, out = pl.pallas_call(
                _mm_res_pool_body,
                grid=(N // RB,),
                in_specs=[
                    pl.BlockSpec((RB, H), lambda r: (r, 0)),
                    pl.BlockSpec((2, RB, HH), lambda r: (0, r, 0)),
                    pl.BlockSpec((H, H), lambda r: (0, 0)),
                    pl.BlockSpec((1, H), lambda r: (0, 0)),
                    pl.BlockSpec((H, H), lambda r: (0, 0)),
                    pl.BlockSpec((1, H), lambda r: (0, 0)),
                ],
                out_specs=[
                    pl.BlockSpec((2, RB, HH), lambda r: (0, r, 0)),
                    pl.BlockSpec((1, 2 * H), lambda r: (0, 0)),
                ],
                out_shape=[
                    jax.ShapeDtypeStruct((2, N, HH), f32),
                    jax.ShapeDtypeStruct((1, 2 * H), f32),
                ],
            )(u, h2.reshape(2, N, HH), mW2, mb2.reshape(1, H),
              W_pool, b_pool.reshape(1, H))

    return out
